# Initial kernel scaffold; baseline (speedup 1.0000x reference)
import jax
import jax.numpy as jnp
from jax.experimental import pallas as pl


def kernel(node, edges, edge_index, W, b):
    raise NotImplementedError("write your pallas kernel here")


# speedup vs baseline: 4.3432x; 4.3432x over previous
import functools

import jax
import jax.numpy as jnp
from jax import lax
from jax.experimental import pallas as pl
from jax.experimental.pallas import tpu as pltpu
from jax.experimental.pallas import tpu_sc as plsc

NC, NS, L = 2, 16, 16
NW = NC * NS
C = 80


def _matmul_bias(node, W, b2d):
    n, f = node.shape
    u = W.shape[1]
    blk = 1000

    def body(x_ref, w_ref, b_ref, o_ref):
        o_ref[...] = (
            jnp.dot(x_ref[...], w_ref[...], preferred_element_type=jnp.float32)
            + b_ref[...]
        )

    return pl.pallas_call(
        body,
        grid=(n // blk,),
        in_specs=[
            pl.BlockSpec((blk, f), lambda i: (i, 0)),
            pl.BlockSpec((f, u), lambda i: (0, 0)),
            pl.BlockSpec((1, u), lambda i: (0, 0)),
        ],
        out_specs=pl.BlockSpec((blk, u), lambda i: (i, 0)),
        out_shape=jax.ShapeDtypeStruct((n, u), jnp.float32),
    )(node, W, b2d)


def _combine_lrelu(partials):
    _, n, u = partials.shape
    blk = 1000

    def body(p_ref, o_ref):
        s = p_ref[0] + p_ref[1]
        o_ref[...] = jnp.where(s > 0, s, 0.2 * s)

    return pl.pallas_call(
        body,
        grid=(n // blk,),
        in_specs=[pl.BlockSpec((2, blk, u), lambda i: (0, i, 0))],
        out_specs=pl.BlockSpec((blk, u), lambda i: (i, 0)),
        out_shape=jax.ShapeDtypeStruct((n, u), jnp.float32),
    )(partials)


def _sc_aggregate(no, src, dst, w):
    e = src.shape[0]
    n, u = no.shape
    epw = e // NW
    nchunks = epw // C
    rq = (n // NS) // 8 * 8
    zrows = 16
    nvec = u // L
    mesh = plsc.VectorSubcoreMesh(core_axis_name="c", subcore_axis_name="s")

    @functools.partial(
        pl.kernel,
        out_type=jax.ShapeDtypeStruct((NC, n, u), jnp.float32),
        mesh=mesh,
        scratch_types=[
            pltpu.VMEM((C,), jnp.int32),
            pltpu.VMEM((C,), jnp.int32),
            pltpu.VMEM((C,), jnp.float32),
            pltpu.VMEM((C, u), jnp.float32),
            pltpu.VMEM((zrows, u), jnp.float32),
            pltpu.VMEM_SHARED((n, u), jnp.float32),
            pltpu.SemaphoreType.DMA,
        ],
    )
    def k(no_hbm, src_hbm, dst_hbm, w_hbm, out_hbm,
          idx_v, dst_v, w_v, rows_v, zero_v, acc, sem):
        cid = lax.axis_index("c")
        sid = lax.axis_index("s")
        wid = cid * NS + sid
        roff = sid * rq
        rtail = n - NS * rq

        zvec = jnp.zeros((L,), jnp.float32)

        def zero_row(i, _):
            for j in range(nvec):
                zero_v[i, pl.ds(j * L, L)] = zvec
            return 0

        lax.fori_loop(0, zrows, zero_row, 0)

        def zero_copy(t, _):
            pltpu.sync_copy(zero_v, acc.at[pl.ds(roff + t * zrows, zrows)])
            return 0

        lax.fori_loop(0, rq // zrows, zero_copy, 0)

        @pl.when(sid == NS - 1)
        def _zero_tail():
            def tail_copy(t, _):
                pltpu.sync_copy(
                    zero_v, acc.at[pl.ds(NS * rq + t * zrows, zrows)]
                )
                return 0

            lax.fori_loop(0, rtail // zrows, tail_copy, 0)

        plsc.subcore_barrier()

        def chunk_body(ci, _):
            base = pl.multiple_of(wid * epw + ci * C, C)
            pltpu.sync_copy(src_hbm.at[pl.ds(base, C)], idx_v)
            pltpu.sync_copy(dst_hbm.at[pl.ds(base, C)], dst_v)
            pltpu.sync_copy(w_hbm.at[pl.ds(base, C)], w_v)
            pltpu.async_copy(no_hbm.at[idx_v], rows_v, sem).wait()

            def group_body(g, _):
                wv16 = w_v[pl.ds(g * L, L)]
                for l in range(L):
                    wv = wv16[l]
                    row = g * L + l
                    for j in range(nvec):
                        sl = pl.ds(j * L, L)
                        rows_v[row, sl] = rows_v[row, sl] * wv
                return 0

            lax.fori_loop(0, C // L, group_body, 0)
            pltpu.sync_copy(rows_v, acc.at[dst_v], add=True)
            return 0

        lax.fori_loop(0, nchunks, chunk_body, 0)
        plsc.subcore_barrier()

        pltpu.sync_copy(
            acc.at[pl.ds(roff, rq)],
            out_hbm.at[cid, pl.ds(roff, rq)],
        )

        @pl.when(sid == NS - 1)
        def _write_tail():
            pltpu.sync_copy(
                acc.at[pl.ds(NS * rq, rtail)],
                out_hbm.at[cid, pl.ds(NS * rq, rtail)],
            )

    return k(no, src, dst, w)


def kernel(node, edges, edge_index, W, b):
    no = _matmul_bias(node, W, b.reshape(1, -1))
    src = edge_index[1]
    dst = edge_index[0]
    w = edges.reshape(-1)
    partials = _sc_aggregate(no, src, dst, w)
    return _combine_lrelu(partials)


# speedup vs baseline: 11.1121x; 2.5585x over previous
import functools

import jax
import jax.numpy as jnp
from jax import lax
from jax.experimental import pallas as pl
from jax.experimental.pallas import tpu as pltpu
from jax.experimental.pallas import tpu_sc as plsc

NC, NS, L = 2, 16, 16
NW = NC * NS
C = 80


def _matmul_bias(node, W, b2d):
    n, f = node.shape
    u = W.shape[1]
    blk = 1000

    def body(x_ref, w_ref, b_ref, o_ref):
        o_ref[...] = (
            jnp.dot(x_ref[...], w_ref[...], preferred_element_type=jnp.float32)
            + b_ref[...]
        )

    return pl.pallas_call(
        body,
        grid=(n // blk,),
        in_specs=[
            pl.BlockSpec((blk, f), lambda i: (i, 0)),
            pl.BlockSpec((f, u), lambda i: (0, 0)),
            pl.BlockSpec((1, u), lambda i: (0, 0)),
        ],
        out_specs=pl.BlockSpec((blk, u), lambda i: (i, 0)),
        out_shape=jax.ShapeDtypeStruct((n, u), jnp.float32),
    )(node, W, b2d)


def _combine_lrelu(partials):
    _, n, u = partials.shape
    blk = 1000

    def body(p_ref, o_ref):
        s = p_ref[0] + p_ref[1]
        o_ref[...] = jnp.where(s > 0, s, 0.2 * s)

    return pl.pallas_call(
        body,
        grid=(n // blk,),
        in_specs=[pl.BlockSpec((2, blk, u), lambda i: (0, i, 0))],
        out_specs=pl.BlockSpec((blk, u), lambda i: (i, 0)),
        out_shape=jax.ShapeDtypeStruct((n, u), jnp.float32),
    )(partials)


NBUF = 4


def _sc_aggregate(no, src, dst, w):
    e = src.shape[0]
    n, u = no.shape
    epw = e // NW
    nchunks = epw // C
    rq = (n // NS) // 8 * 8
    rtail = n - NS * rq
    nzfull = rq // C
    zrem = rq - nzfull * C
    nvec = u // L
    mesh = plsc.VectorSubcoreMesh(core_axis_name="c", subcore_axis_name="s")

    @functools.partial(
        pl.kernel,
        out_type=jax.ShapeDtypeStruct((NC, n, u), jnp.float32),
        mesh=mesh,
        scratch_types=[
            pltpu.VMEM((NBUF, C), jnp.int32),
            pltpu.VMEM((NBUF, C), jnp.int32),
            pltpu.VMEM((NBUF, C), jnp.float32),
            pltpu.VMEM((NBUF, C, u), jnp.float32),
            pltpu.VMEM_SHARED((n, u), jnp.float32),
            pltpu.SemaphoreType.DMA((NBUF,)),
            pltpu.SemaphoreType.DMA((NBUF,)),
            pltpu.SemaphoreType.DMA,
        ],
    )
    def k(no_hbm, src_hbm, dst_hbm, w_hbm, out_hbm,
          src_r, dst_r, w_r, rows, acc, sems, ssems, zsem):
        cid = lax.axis_index("c")
        sid = lax.axis_index("s")
        wid = cid * NS + sid
        roff = sid * rq
        ebase = pl.multiple_of(wid * epw, 8)

        def chunk_off(ci):
            return pl.multiple_of(ebase + ci * C, 8)

        def issue_src(ci, b):
            pltpu.async_copy(
                src_hbm.at[pl.ds(chunk_off(ci), C)], src_r.at[b], ssems.at[b]
            )

        def issue_main(ci, b):
            base = chunk_off(ci)
            pltpu.async_copy(no_hbm.at[src_r.at[b]], rows.at[b], sems.at[b])
            pltpu.async_copy(
                dst_hbm.at[pl.ds(base, C)], dst_r.at[b], sems.at[b]
            )
            pltpu.async_copy(w_hbm.at[pl.ds(base, C)], w_r.at[b], sems.at[b])

        def wait_main(ci, b):
            pltpu.make_async_copy(
                no_hbm.at[src_r.at[b]], rows.at[b], sems.at[b]
            ).wait()
            pltpu.make_async_copy(
                dst_hbm.at[pl.ds(chunk_off(ci), C)], dst_r.at[b], sems.at[b]
            ).wait()
            pltpu.make_async_copy(
                w_hbm.at[pl.ds(chunk_off(ci), C)], w_r.at[b], sems.at[b]
            ).wait()

        def scale_rows(b):
            def group_body(g, _):
                wv16 = w_r[b, pl.ds(g * L, L)]
                for l in range(L):
                    wv = wv16[l]
                    row = g * L + l
                    for j in range(nvec):
                        sl = pl.ds(j * L, L)
                        rows[b, row, sl] = rows[b, row, sl] * wv
                return 0

            lax.fori_loop(0, C // L, group_body, 0)

        zvec = jnp.zeros((L,), jnp.float32)

        def zero_row(i, _):
            for j in range(nvec):
                rows[0, i, pl.ds(j * L, L)] = zvec
            return 0

        lax.fori_loop(0, C, zero_row, 0)
        zdescs = [
            pltpu.async_copy(
                rows.at[0], acc.at[pl.ds(roff + t * C, C)], zsem
            )
            for t in range(nzfull)
        ]
        zdescs.append(
            pltpu.async_copy(
                rows.at[0, pl.ds(0, zrem)],
                acc.at[pl.ds(roff + nzfull * C, zrem)],
                zsem,
            )
        )
        for b in range(NBUF):
            issue_src(b, b)
        for d in zdescs:
            d.wait()

        @pl.when(sid == NS - 1)
        def _zero_tail():
            pltpu.sync_copy(
                rows.at[0, pl.ds(0, rtail)], acc.at[pl.ds(NS * rq, rtail)]
            )

        plsc.subcore_barrier()

        for b in range(NBUF):
            pltpu.make_async_copy(
                src_hbm.at[pl.ds(chunk_off(b), C)], src_r.at[b], ssems.at[b]
            ).wait()
            issue_main(b, b)

        def process(ci, b, lookahead):
            wait_main(ci, b)
            if lookahead:
                @pl.when(ci + NBUF < nchunks)
                def _src_next():
                    issue_src(ci + NBUF, b)
            scale_rows(b)
            pltpu.sync_copy(rows.at[b], acc.at[dst_r.at[b]], add=True)
            if lookahead:
                @pl.when(ci + NBUF < nchunks)
                def _main_next():
                    pltpu.make_async_copy(
                        src_hbm.at[pl.ds(chunk_off(ci + NBUF), C)],
                        src_r.at[b],
                        ssems.at[b],
                    ).wait()
                    issue_main(ci + NBUF, b)

        def outer_body(r, _):
            for b in range(NBUF):
                process(r * NBUF + b, b, True)
            return 0

        nfull = nchunks // NBUF
        lax.fori_loop(0, nfull, outer_body, 0)
        for ci in range(nfull * NBUF, nchunks):
            process(ci, ci - nfull * NBUF, False)
        plsc.subcore_barrier()

        pltpu.sync_copy(
            acc.at[pl.ds(roff, rq)],
            out_hbm.at[cid, pl.ds(roff, rq)],
        )

        @pl.when(sid == NS - 1)
        def _write_tail():
            pltpu.sync_copy(
                acc.at[pl.ds(NS * rq, rtail)],
                out_hbm.at[cid, pl.ds(NS * rq, rtail)],
            )

    return k(no, src, dst, w)


def kernel(node, edges, edge_index, W, b):
    no = _matmul_bias(node, W, b.reshape(1, -1))
    partials = _sc_aggregate(
        no, edge_index[1], edge_index[0], edges.reshape(-1)
    )
    return _combine_lrelu(partials)
